# baseline (speedup 1.0000x reference)
import functools

import jax
import jax.numpy as jnp
from jax import lax
from jax.experimental import pallas as pl
from jax.experimental.pallas import tpu as pltpu
from jax.experimental.pallas import tpu_sc as plsc

_NC, _NS, _L = 2, 16, 16
_NW = _NC * _NS
_NB = 1024
_NG = _NB // _L

_KEEP_RATIO_HIGH = 0.25
_UNROLL = 16
_CUNROLL = 4


def _sc_topk_mask_body(k, t, rows_per_w,
                       probs_hbm, out_hbm,
                       data_v, cbits_v, cidx_v, hist_v, btot_v, gtot_s):
    wid = lax.axis_index("s") * _NC + lax.axis_index("c")
    lanes = lax.broadcasted_iota(jnp.int32, (_L,), 0)
    lanes_nb = lanes * _NB
    ones = jnp.ones((_L,), jnp.int32)
    zeros = jnp.zeros((_L,), jnp.int32)
    n_chunks = t // _L

    def load_bits(ref, base):
        return lax.bitcast_convert_type(ref[pl.ds(base, _L)], jnp.int32)

    def splat_to_scalar(v):
        return jnp.max(v)

    def consume_hist(kneed, ngroups, from_top):
        def _cg(c, carry):
            acc = zeros
            for r in range(_L):
                sl = pl.ds(r * _NB + c * _L, _L)
                acc = acc + hist_v[sl]
                hist_v[sl] = zeros
            btot_v[pl.ds(c * _L, _L)] = acc
            gtot_s[c] = jnp.sum(acc)
            return carry
        lax.fori_loop(0, ngroups, _cg, 0)

        z = jnp.int32(0)

        def sg(i, cy):
            cum, gstar, cat, found = cy
            g = (ngroups - 1 - i) if from_top else i
            tt = gtot_s[g]
            hit = jnp.logical_and(found == 0, cum + tt >= kneed)
            gstar = jnp.where(hit, g, gstar)
            cat = jnp.where(hit, cum, cat)
            found = jnp.where(hit, 1, found)
            return (cum + tt, gstar, cat, found)

        _, gstar, cat, _ = lax.fori_loop(0, ngroups, sg, (z, z, z, z))

        hv = btot_v[pl.ds(gstar * _L, _L)]
        sv = lax.rev(hv, (0,)) if from_top else hv
        cs = plsc.cumsum(sv) + cat
        fs = plsc.all_reduce_ffs(cs >= kneed)
        jpos = splat_to_scalar(fs) if fs.ndim else fs
        before = jnp.sum(jnp.where(lanes == jpos, cs - sv, 0))
        b_in = (15 - jpos) if from_top else jpos
        return gstar * _L + b_in, before

    def cand_hist_pass(src_v, n, bucket_fn, mask_fn):
        def _hb(i, c):
            bases = [(i * _CUNROLL + u) * _L for u in range(_CUNROLL)]
            bs = [src_v[pl.ds(base, _L)] for base in bases]
            for base, b in zip(bases, bs):
                valid = (base + lanes) < n
                m = mask_fn(b)
                m = valid if m is None else jnp.logical_and(valid, m)
                plsc.addupdate_scatter(hist_v, [lanes_nb + bucket_fn(b)],
                                       ones, mask=m)
            return c
        lax.fori_loop(0, (n + _L * _CUNROLL - 1) // (_L * _CUNROLL), _hb, 0)

    def row_body(r, carry):
        row = wid * rows_per_w + r
        pltpu.sync_copy(probs_hbm.at[row], data_v)

        def _h1(i, c):
            bs = [load_bits(data_v, (i * _UNROLL + u) * _L)
                  for u in range(_UNROLL)]
            for b in bs:
                plsc.addupdate_scatter(
                    hist_v, [lanes_nb + lax.shift_right_logical(b, 20)], ones)
            return c
        lax.fori_loop(0, n_chunks // _UNROLL, _h1, 0)

        b1, above1 = consume_hist(k, _NG, True)
        k1 = k - above1

        @plsc.parallel_loop(0, n_chunks, unroll=_UNROLL)
        def _cpa(i):
            base = i * _L
            b = load_bits(data_v, base)
            m = lax.shift_right_logical(b, 20) == b1
            cidx_v[pl.ds(base, _L)] = plsc.cumsum(m.astype(jnp.int32))

        def cp(i, offv):
            bases = [(i * _UNROLL + u) * _L for u in range(_UNROLL)]
            bs = [load_bits(data_v, base) for base in bases]
            pcs = [cidx_v[pl.ds(base, _L)] for base in bases]
            for base, b, pc in zip(bases, bs, pcs):
                m = lax.shift_right_logical(b, 20) == b1
                dest = offv + pc - 1
                plsc.store_scatter(cbits_v, [dest], b, mask=m)
                plsc.store_scatter(cidx_v, [dest], base + lanes, mask=m)
                offv = offv + plsc.all_reduce_population_count(m)
            return offv
        n1 = splat_to_scalar(
            lax.fori_loop(0, n_chunks // _UNROLL, cp, zeros))

        cand_hist_pass(cbits_v, n1,
                       lambda b: lax.shift_right_logical(b, 10) & 1023,
                       lambda b: None)
        b2, above2 = consume_hist(k1, _NG, True)
        k2 = k1 - above2

        def cp2(i, offv):
            base = i * _L
            b = cbits_v[pl.ds(base, _L)]
            iv = cidx_v[pl.ds(base, _L)]
            valid = (base + lanes) < n1
            m = jnp.logical_and(
                valid, (lax.shift_right_logical(b, 10) & 1023) == b2)
            dest = offv + plsc.cumsum(m.astype(jnp.int32)) - 1
            plsc.store_scatter(cbits_v, [dest], b, mask=m)
            plsc.store_scatter(cidx_v, [dest], iv, mask=m)
            return offv + plsc.all_reduce_population_count(m)
        n2 = splat_to_scalar(
            lax.fori_loop(0, (n1 + _L - 1) // _L, cp2, zeros))

        def fast_path(_):
            b = cbits_v[pl.ds(0, _L)]
            iv = cidx_v[pl.ds(0, _L)]
            valid = lanes < n2
            bpad = jnp.where(valid, b, jnp.int32(-2147483648))
            skeys, _ = plsc.sort_key_val(bpad, iv, descending=True)
            thr_f = jnp.sum(jnp.where(lanes == k2 - 1, skeys, 0))
            meq = jnp.logical_and(valid, b == thr_f)
            need_t = k2 - jnp.sum(
                jnp.where(jnp.logical_and(valid, b > thr_f), 1, 0))
            cms = plsc.cumsum(meq.astype(jnp.int32))
            hit = jnp.logical_and(meq, cms == need_t)
            istar_f = jnp.sum(jnp.where(hit, iv, 0))
            return thr_f, istar_f

        def slow_path(_):
            cand_hist_pass(cbits_v, n2, lambda b: b & 1023, lambda b: None)
            b3, above3 = consume_hist(k2, _NG, True)
            need = k2 - above3
            thr_s = (((b1 << 10) | b2) << 10) | b3

            def cp3(i, offv):
                base = i * _L
                b = cbits_v[pl.ds(base, _L)]
                iv = cidx_v[pl.ds(base, _L)]
                valid = (base + lanes) < n2
                m = jnp.logical_and(valid, (b & 1023) == b3)
                dest = offv + plsc.cumsum(m.astype(jnp.int32)) - 1
                plsc.store_scatter(cidx_v, [dest], iv, mask=m)
                return offv + plsc.all_reduce_population_count(m)
            n3 = splat_to_scalar(
                lax.fori_loop(0, (n2 + _L - 1) // _L, cp3, zeros))

            cand_hist_pass(cidx_v, n3,
                           lambda iv: lax.shift_right_logical(iv, 5),
                           lambda iv: None)
            ib1, below1 = consume_hist(need, _NG, False)
            need2 = need - below1

            cand_hist_pass(cidx_v, n3, lambda iv: iv & 31,
                           lambda iv: lax.shift_right_logical(iv, 5) == ib1)
            ib2, _ = consume_hist(need2, 2, False)
            return thr_s, (ib1 << 5) | ib2

        thr, istar = lax.cond(n2 <= _L, fast_path, slow_path, 0)

        @plsc.parallel_loop(0, n_chunks, unroll=_UNROLL)
        def _mb(i):
            base = i * _L
            b = load_bits(data_v, base)
            iv = base + lanes
            keep = jnp.logical_or(
                b > thr, jnp.logical_and(b == thr, iv <= istar))
            data_v[pl.ds(base, _L)] = jnp.where(
                keep, jnp.float32(1.0), jnp.float32(0.0))

        pltpu.sync_copy(data_v, out_hbm.at[row])
        return carry

    @plsc.parallel_loop(0, _NB, unroll=_UNROLL)
    def _zb(j):
        hist_v[pl.ds(j * _L, _L)] = zeros

    lax.fori_loop(0, rows_per_w, row_body, 0)


def kernel(probs):
    b, t = probs.shape
    k = min(max(1, int(t * _KEEP_RATIO_HIGH)), t)
    rows_per_w = b // _NW
    pad = _L * (_UNROLL + 1)
    mesh = plsc.VectorSubcoreMesh(core_axis_name="c", subcore_axis_name="s",
                                  num_cores=_NC, num_subcores=_NS)
    f = pl.kernel(
        functools.partial(_sc_topk_mask_body, k, t, rows_per_w),
        out_type=jax.ShapeDtypeStruct((b, t), jnp.float32),
        mesh=mesh,
        compiler_params=pltpu.CompilerParams(needs_layout_passes=False),
        scratch_types=[
            pltpu.VMEM((t,), jnp.float32),
            pltpu.VMEM((t + pad,), jnp.int32),
            pltpu.VMEM((t + pad,), jnp.int32),
            pltpu.VMEM((_L * _NB,), jnp.int32),
            pltpu.VMEM((_NB,), jnp.int32),
            pltpu.SMEM((_NG,), jnp.int32),
        ],
    )
    return f(probs)


# speedup vs baseline: 1.3049x; 1.3049x over previous
import functools

import jax
import jax.numpy as jnp
from jax import lax
from jax.experimental import pallas as pl
from jax.experimental.pallas import tpu as pltpu
from jax.experimental.pallas import tpu_sc as plsc

_NC, _NS, _L = 2, 16, 16
_NW = _NC * _NS
_NB = 1024
_NG = _NB // _L

_KEEP_RATIO_HIGH = 0.25
_UNROLL = 8
_CUNROLL = 4


def _sc_topk_mask_body(k, t, rows_per_w,
                       probs_hbm, out_hbm,
                       data_v, cbits_v, cidx_v, hist_v, btot_v, gtot_s):
    wid = lax.axis_index("s") * _NC + lax.axis_index("c")
    lanes = lax.broadcasted_iota(jnp.int32, (_L,), 0)
    lanes_nb = lanes * _NB
    ones = jnp.ones((_L,), jnp.int32)
    zeros = jnp.zeros((_L,), jnp.int32)
    n_chunks = t // _L

    def load_bits(ref, base):
        return lax.bitcast_convert_type(ref[pl.ds(base, _L)], jnp.int32)

    def splat_to_scalar(v):
        return jnp.max(v)

    def consume_hist(kneed, ngroups, from_top):
        def _cg(c, carry):
            acc = zeros
            for r in range(_L):
                sl = pl.ds(r * _NB + c * _L, _L)
                acc = acc + hist_v[sl]
                hist_v[sl] = zeros
            btot_v[pl.ds(c * _L, _L)] = acc
            gtot_s[c] = jnp.sum(acc)
            return carry
        lax.fori_loop(0, ngroups, _cg, 0)

        z = jnp.int32(0)

        def sg(i, cy):
            cum, gstar, cat, found = cy
            g = (ngroups - 1 - i) if from_top else i
            tt = gtot_s[g]
            hit = jnp.logical_and(found == 0, cum + tt >= kneed)
            gstar = jnp.where(hit, g, gstar)
            cat = jnp.where(hit, cum, cat)
            found = jnp.where(hit, 1, found)
            return (cum + tt, gstar, cat, found)

        _, gstar, cat, _ = lax.fori_loop(0, ngroups, sg, (z, z, z, z))

        hv = btot_v[pl.ds(gstar * _L, _L)]
        sv = lax.rev(hv, (0,)) if from_top else hv
        cs = plsc.cumsum(sv) + cat
        fs = plsc.all_reduce_ffs(cs >= kneed)
        jpos = splat_to_scalar(fs) if fs.ndim else fs
        before = jnp.sum(jnp.where(lanes == jpos, cs - sv, 0))
        b_in = (15 - jpos) if from_top else jpos
        return gstar * _L + b_in, before

    def cand_hist_pass(src_v, n, bucket_fn, mask_fn):
        def _hb(i, c):
            bases = [(i * _CUNROLL + u) * _L for u in range(_CUNROLL)]
            bs = [src_v[pl.ds(base, _L)] for base in bases]
            for base, b in zip(bases, bs):
                valid = (base + lanes) < n
                m = mask_fn(b)
                m = valid if m is None else jnp.logical_and(valid, m)
                plsc.addupdate_scatter(hist_v, [lanes_nb + bucket_fn(b)],
                                       ones, mask=m)
            return c
        lax.fori_loop(0, (n + _L * _CUNROLL - 1) // (_L * _CUNROLL), _hb, 0)

    def row_body(r, carry):
        row = wid * rows_per_w + r
        pltpu.sync_copy(probs_hbm.at[row], data_v)

        def _h1(i, c):
            bs = [load_bits(data_v, (i * _UNROLL + u) * _L)
                  for u in range(_UNROLL)]
            for b in bs:
                plsc.addupdate_scatter(
                    hist_v, [lanes_nb + lax.shift_right_logical(b, 20)], ones)
            return c
        lax.fori_loop(0, n_chunks // _UNROLL, _h1, 0)

        b1, above1 = consume_hist(k, _NG, True)
        k1 = k - above1

        @plsc.parallel_loop(0, n_chunks, unroll=_UNROLL)
        def _cpa(i):
            base = i * _L
            b = load_bits(data_v, base)
            m = lax.shift_right_logical(b, 20) == b1
            cidx_v[pl.ds(base, _L)] = plsc.cumsum(m.astype(jnp.int32))

        def cp(i, offv):
            bases = [(i * _UNROLL + u) * _L for u in range(_UNROLL)]
            bs = [load_bits(data_v, base) for base in bases]
            pcs = [cidx_v[pl.ds(base, _L)] for base in bases]
            for base, b, pc in zip(bases, bs, pcs):
                m = lax.shift_right_logical(b, 20) == b1
                dest = offv + pc - 1
                plsc.store_scatter(cbits_v, [dest], b, mask=m)
                plsc.store_scatter(cidx_v, [dest], base + lanes, mask=m)
                offv = offv + plsc.all_reduce_population_count(m)
            return offv
        n1 = splat_to_scalar(
            lax.fori_loop(0, n_chunks // _UNROLL, cp, zeros))

        cand_hist_pass(cbits_v, n1,
                       lambda b: lax.shift_right_logical(b, 10) & 1023,
                       lambda b: None)
        b2, above2 = consume_hist(k1, _NG, True)
        k2 = k1 - above2

        def cp2(i, offv):
            base = i * _L
            b = cbits_v[pl.ds(base, _L)]
            iv = cidx_v[pl.ds(base, _L)]
            valid = (base + lanes) < n1
            m = jnp.logical_and(
                valid, (lax.shift_right_logical(b, 10) & 1023) == b2)
            dest = offv + plsc.cumsum(m.astype(jnp.int32)) - 1
            plsc.store_scatter(cbits_v, [dest], b, mask=m)
            plsc.store_scatter(cidx_v, [dest], iv, mask=m)
            return offv + plsc.all_reduce_population_count(m)
        n2 = splat_to_scalar(
            lax.fori_loop(0, (n1 + _L - 1) // _L, cp2, zeros))

        def fast_path(_):
            b = cbits_v[pl.ds(0, _L)]
            iv = cidx_v[pl.ds(0, _L)]
            valid = lanes < n2
            bpad = jnp.where(valid, b, jnp.int32(-2147483648))
            skeys, _ = plsc.sort_key_val(bpad, iv, descending=True)
            thr_f = jnp.sum(jnp.where(lanes == k2 - 1, skeys, 0))
            meq = jnp.logical_and(valid, b == thr_f)
            need_t = k2 - jnp.sum(
                jnp.where(jnp.logical_and(valid, b > thr_f), 1, 0))
            cms = plsc.cumsum(meq.astype(jnp.int32))
            hit = jnp.logical_and(meq, cms == need_t)
            istar_f = jnp.sum(jnp.where(hit, iv, 0))
            return thr_f, istar_f

        def slow_path(_):
            cand_hist_pass(cbits_v, n2, lambda b: b & 1023, lambda b: None)
            b3, above3 = consume_hist(k2, _NG, True)
            need = k2 - above3
            thr_s = (((b1 << 10) | b2) << 10) | b3

            def cp3(i, offv):
                base = i * _L
                b = cbits_v[pl.ds(base, _L)]
                iv = cidx_v[pl.ds(base, _L)]
                valid = (base + lanes) < n2
                m = jnp.logical_and(valid, (b & 1023) == b3)
                dest = offv + plsc.cumsum(m.astype(jnp.int32)) - 1
                plsc.store_scatter(cidx_v, [dest], iv, mask=m)
                return offv + plsc.all_reduce_population_count(m)
            n3 = splat_to_scalar(
                lax.fori_loop(0, (n2 + _L - 1) // _L, cp3, zeros))

            cand_hist_pass(cidx_v, n3,
                           lambda iv: lax.shift_right_logical(iv, 5),
                           lambda iv: None)
            ib1, below1 = consume_hist(need, _NG, False)
            need2 = need - below1

            cand_hist_pass(cidx_v, n3, lambda iv: iv & 31,
                           lambda iv: lax.shift_right_logical(iv, 5) == ib1)
            ib2, _ = consume_hist(need2, 2, False)
            return thr_s, (ib1 << 5) | ib2

        thr, istar = lax.cond(n2 <= _L, fast_path, slow_path, 0)

        @plsc.parallel_loop(0, n_chunks, unroll=_UNROLL)
        def _mb(i):
            base = i * _L
            b = load_bits(data_v, base)
            iv = base + lanes
            keep = jnp.logical_or(
                b > thr, jnp.logical_and(b == thr, iv <= istar))
            data_v[pl.ds(base, _L)] = jnp.where(
                keep, jnp.float32(1.0), jnp.float32(0.0))

        pltpu.sync_copy(data_v, out_hbm.at[row])
        return carry

    @plsc.parallel_loop(0, _NB, unroll=_UNROLL)
    def _zb(j):
        hist_v[pl.ds(j * _L, _L)] = zeros

    lax.fori_loop(0, rows_per_w, row_body, 0)


def kernel(probs):
    b, t = probs.shape
    k = min(max(1, int(t * _KEEP_RATIO_HIGH)), t)
    rows_per_w = b // _NW
    pad = _L * (_UNROLL + 1)
    mesh = plsc.VectorSubcoreMesh(core_axis_name="c", subcore_axis_name="s",
                                  num_cores=_NC, num_subcores=_NS)
    f = pl.kernel(
        functools.partial(_sc_topk_mask_body, k, t, rows_per_w),
        out_type=jax.ShapeDtypeStruct((b, t), jnp.float32),
        mesh=mesh,
        compiler_params=pltpu.CompilerParams(needs_layout_passes=False),
        scratch_types=[
            pltpu.VMEM((t,), jnp.float32),
            pltpu.VMEM((t + pad,), jnp.int32),
            pltpu.VMEM((t + pad,), jnp.int32),
            pltpu.VMEM((_L * _NB,), jnp.int32),
            pltpu.VMEM((_NB,), jnp.int32),
            pltpu.SMEM((_NG,), jnp.int32),
        ],
    )
    return f(probs)
